# baseline (speedup 1.0000x reference)
import jax
import jax.numpy as jnp
from jax import lax
from jax.experimental import pallas as pl
from jax.experimental.pallas import tpu as pltpu, tpu_sc as plsc

_B, _C, _T = 4, 1024, 8192
_INFO = plsc.get_sparse_core_info()
_NW = _INFO.num_cores * _INFO.num_subcores
_NCB = 16
_CC = _C // _NCB
_NTS = _NW // _NCB
_TS = _T // _NTS
_TC = 128
_NCHUNK = _TS // _TC
_NJ = _TC // 16


def _sc_body(x_hbm, table_hbm, out_hbm, *scratch):
    xbufs = list(scratch[0:4])
    tbuf = scratch[4]
    xin = list(scratch[5:9])
    xout = list(scratch[9:13])

    wid = lax.axis_index("s") * _INFO.num_cores + lax.axis_index("c")
    cb = wid % _NCB
    c0x = cb * _CC
    c0t = (cb // 2) * 128
    coff = (cb % 2) * _CC
    ts0 = (wid // _NCB) * _TS

    iota16 = lax.iota(jnp.int32, 16)

    def compute(xb):
        @plsc.parallel_loop(0, _CC * _NJ, unroll=8)
        def _(i):
            c = i >> 3
            joff = (i & (_NJ - 1)) * 16
            t_idx = iota16 + joff
            cvec = jnp.full((16,), coff + c, jnp.int32)
            tv = plsc.load_gather(tbuf, [t_idx, cvec])
            plsc.addupdate(xb.at[c, pl.ds(joff, 16)], tv)

    def chunk_body(tc, _):
        t0 = ts0 + tc * _TC
        pltpu.sync_copy(table_hbm.at[pl.ds(t0, _TC), pl.ds(c0t, 128)], tbuf)
        incopies = [
            pltpu.make_async_copy(
                x_hbm.at[b, pl.ds(c0x, _CC), pl.ds(t0, _TC)], xbufs[b], xin[b])
            for b in range(_B)
        ]
        outcopies = [
            pltpu.make_async_copy(
                xbufs[b], out_hbm.at[b, pl.ds(c0x, _CC), pl.ds(t0, _TC)],
                xout[b])
            for b in range(_B)
        ]
        for b in range(_B):
            incopies[b].start()
        for b in range(_B):
            incopies[b].wait()
            compute(xbufs[b])
            outcopies[b].start()
        for b in range(_B):
            outcopies[b].wait()
        return 0

    lax.fori_loop(0, _NCHUNK, chunk_body, 0)


def kernel(x, table):
    mesh = plsc.VectorSubcoreMesh(core_axis_name="c", subcore_axis_name="s")
    run = pl.kernel(
        _sc_body,
        out_type=jax.ShapeDtypeStruct((_B, _C, _T), jnp.float32),
        mesh=mesh,
        compiler_params=pltpu.CompilerParams(needs_layout_passes=False),
        scratch_types=(
            [pltpu.VMEM((_CC, _TC), jnp.float32) for _ in range(4)]
            + [pltpu.VMEM((_TC, 128), jnp.float32)]
            + [pltpu.SemaphoreType.DMA for _ in range(8)]
        ),
    )
    return run(x, table)


# speedup vs baseline: 1.3050x; 1.3050x over previous
import jax
import jax.numpy as jnp
from jax import lax
from jax.experimental import pallas as pl
from jax.experimental.pallas import tpu as pltpu, tpu_sc as plsc

_B, _C, _T = 4, 1024, 8192
_INFO = plsc.get_sparse_core_info()
_NW = _INFO.num_cores * _INFO.num_subcores
_NCB = 16
_CC = _C // _NCB
_NTS = _NW // _NCB
_TS = _T // _NTS
_TC = 128
_NCHUNK = _TS // _TC
_NJ = _TC // 16


def _sc_body(x_hbm, table_hbm, out_hbm, *scratch):
    xbufs = list(scratch[0:4])
    tbuf = scratch[4]
    xin = list(scratch[5:9])
    xout = list(scratch[9:13])

    wid = lax.axis_index("s") * _INFO.num_cores + lax.axis_index("c")
    cb = wid % _NCB
    c0x = cb * _CC
    c0t = (cb // 2) * 128
    coff = (cb % 2) * _CC
    ts0 = (wid // _NCB) * _TS

    tidx = [lax.iota(jnp.int32, 16) + j * 16 for j in range(_NJ)]

    def compute(xb):
        @plsc.parallel_loop(0, _CC, unroll=4)
        def _(c):
            cvec = jnp.full((16,), coff + c, jnp.int32)
            for j in range(_NJ):
                tv = plsc.load_gather(tbuf, [tidx[j], cvec])
                plsc.addupdate(xb.at[c, pl.ds(j * 16, 16)], tv)

    def chunk_body(tc, _):
        t0 = ts0 + tc * _TC
        pltpu.sync_copy(table_hbm.at[pl.ds(t0, _TC), pl.ds(c0t, 128)], tbuf)
        incopies = [
            pltpu.make_async_copy(
                x_hbm.at[b, pl.ds(c0x, _CC), pl.ds(t0, _TC)], xbufs[b], xin[b])
            for b in range(_B)
        ]
        outcopies = [
            pltpu.make_async_copy(
                xbufs[b], out_hbm.at[b, pl.ds(c0x, _CC), pl.ds(t0, _TC)],
                xout[b])
            for b in range(_B)
        ]
        for b in range(_B):
            incopies[b].start()
        for b in range(_B):
            incopies[b].wait()
            compute(xbufs[b])
            outcopies[b].start()
        for b in range(_B):
            outcopies[b].wait()
        return 0

    lax.fori_loop(0, _NCHUNK, chunk_body, 0)


def kernel(x, table):
    mesh = plsc.VectorSubcoreMesh(core_axis_name="c", subcore_axis_name="s")
    run = pl.kernel(
        _sc_body,
        out_type=jax.ShapeDtypeStruct((_B, _C, _T), jnp.float32),
        mesh=mesh,
        compiler_params=pltpu.CompilerParams(needs_layout_passes=False),
        scratch_types=(
            [pltpu.VMEM((_CC, _TC), jnp.float32) for _ in range(4)]
            + [pltpu.VMEM((_TC, 128), jnp.float32)]
            + [pltpu.SemaphoreType.DMA for _ in range(8)]
        ),
    )
    return run(x, table)


# speedup vs baseline: 1.3056x; 1.0004x over previous
import jax
import jax.numpy as jnp
from jax import lax
from jax.experimental import pallas as pl
from jax.experimental.pallas import tpu as pltpu, tpu_sc as plsc

_B, _C, _T = 4, 1024, 8192
_INFO = plsc.get_sparse_core_info()
_NW = _INFO.num_cores * _INFO.num_subcores
_NCB = 16
_CC = _C // _NCB
_NTS = _NW // _NCB
_TS = _T // _NTS
_TC = 128
_NCHUNK = _TS // _TC
_NJ = _TC // 16


def _sc_body(x_hbm, table_hbm, out_hbm, *scratch):
    xbufs = list(scratch[0:4])
    tbuf = scratch[4]
    xin = list(scratch[5:9])
    xout = list(scratch[9:13])

    wid = lax.axis_index("s") * _INFO.num_cores + lax.axis_index("c")
    cb = wid % _NCB
    c0x = cb * _CC
    c0t = (cb // 2) * 128
    coff = (cb % 2) * _CC
    ts0 = (wid // _NCB) * _TS

    tidx = [lax.iota(jnp.int32, 16) + j * 16 for j in range(_NJ)]

    def compute(xb):
        @plsc.parallel_loop(0, _CC, unroll=4)
        def _(c):
            cvec = jnp.full((16,), coff + c, jnp.int32)
            for j in range(_NJ):
                tv = plsc.load_gather(tbuf, [tidx[j], cvec])
                plsc.addupdate(xb.at[c, pl.ds(j * 16, 16)], tv)

    def chunk_body(tc, _):
        t0 = ts0 + tc * _TC
        pltpu.sync_copy(table_hbm.at[pl.ds(t0, _TC), pl.ds(c0t, 128)],
                        tbuf.at[:, pl.ds(0, 128)])
        incopies = [
            pltpu.make_async_copy(
                x_hbm.at[b, pl.ds(c0x, _CC), pl.ds(t0, _TC)], xbufs[b], xin[b])
            for b in range(_B)
        ]
        outcopies = [
            pltpu.make_async_copy(
                xbufs[b], out_hbm.at[b, pl.ds(c0x, _CC), pl.ds(t0, _TC)],
                xout[b])
            for b in range(_B)
        ]
        for b in range(_B):
            incopies[b].start()
        for b in range(_B):
            incopies[b].wait()
            compute(xbufs[b])
            outcopies[b].start()
        for b in range(_B):
            outcopies[b].wait()
        return 0

    lax.fori_loop(0, _NCHUNK, chunk_body, 0)


def kernel(x, table):
    mesh = plsc.VectorSubcoreMesh(core_axis_name="c", subcore_axis_name="s")
    run = pl.kernel(
        _sc_body,
        out_type=jax.ShapeDtypeStruct((_B, _C, _T), jnp.float32),
        mesh=mesh,
        compiler_params=pltpu.CompilerParams(needs_layout_passes=False),
        scratch_types=(
            [pltpu.VMEM((_CC, _TC), jnp.float32) for _ in range(4)]
            + [pltpu.VMEM((_TC, 129), jnp.float32)]
            + [pltpu.SemaphoreType.DMA for _ in range(8)]
        ),
    )
    return run(x, table)


# speedup vs baseline: 8.1279x; 6.2254x over previous
import jax
import jax.numpy as jnp
from jax.experimental import pallas as pl


def _body(x_ref, t_ref, o_ref):
    o_ref[...] = x_ref[...] + jnp.transpose(t_ref[...])[None]


def kernel(x, table):
    B, C, T = x.shape
    Tb = 512
    return pl.pallas_call(
        _body,
        grid=(T // Tb,),
        in_specs=[
            pl.BlockSpec((B, C, Tb), lambda i: (0, 0, i)),
            pl.BlockSpec((Tb, C), lambda i: (i, 0)),
        ],
        out_specs=pl.BlockSpec((B, C, Tb), lambda i: (0, 0, i)),
        out_shape=jax.ShapeDtypeStruct((B, C, T), x.dtype),
    )(x, table)
